# baseline (speedup 1.0000x reference)
import jax
import jax.numpy as jnp
from jax import lax
from jax.experimental import pallas as pl
from jax.experimental.pallas import tpu as pltpu
from jax.experimental.pallas import tpu_sc as plsc

F32 = jnp.float32
EPS = 1e-5
NC = 2
NS = 16
NW = NC * NS
LANES = 16
CH = 80
BE = 2560


def _sc_mesh():
    return plsc.VectorSubcoreMesh(
        core_axis_name="c", subcore_axis_name="s",
        num_cores=NC, num_subcores=NS)


def _fill_zero(buf, rows, cols):
    zero = jnp.zeros((LANES,), F32)

    def body(r, carry):
        for j in range(cols // LANES):
            buf[r, pl.ds(j * LANES, LANES)] = zero
        return carry

    lax.fori_loop(0, rows, body, 0)


def _zero_acc_rows(src_buf, rows, acc, base, nrows):
    nfull, rem = divmod(nrows, rows)
    for k in range(nfull):
        pltpu.sync_copy(src_buf, acc.at[pl.ds(base + k * rows, rows)])
    if rem:
        pltpu.sync_copy(src_buf.at[pl.ds(0, rem)],
                        acc.at[pl.ds(base + nfull * rows, rem)])


def _pad_rows(N):
    nt = -(-N // NS)
    nt = -(-nt // 8) * 8
    return nt * NS, nt


CG = 64


def _gather_geometry(N, E):
    QTOT = E // CG
    QMAIN = QTOT // NW
    XTRA = QTOT - QMAIN * NW
    NQ = QMAIN + 1
    NQ += NQ % 2
    NQPAD = -(-NQ // 8) * 8
    return QTOT, QMAIN, XTRA, NQ, NQPAD


def _make_gather_segsum(N, E, D):
    QTOT, QMAIN, XTRA, NQ, NQPAD = _gather_geometry(N, E)
    NP, NT = _pad_rows(N)
    T = NQ // 2
    MW = QMAIN * CG

    def body(atom, src3, dst3, sumh, part, src_v, dst_v,
             as0, ad0, as1, ad1, acc, sg0, sg1, swc):
        c = lax.axis_index("c")
        s = lax.axis_index("s")
        wid = c * NS + s
        _fill_zero(ad0, CG, D)
        _zero_acc_rows(ad0, CG, acc, s * NT, NT)
        plsc.subcore_barrier()

        def ebase(q):
            main = wid * MW + q * CG
            extra = QMAIN * NW * CG + wid * CG
            sink = E + wid * CG
            is_extra = jnp.logical_and(q == QMAIN, wid < XTRA)
            return jnp.where(q < QMAIN, main, jnp.where(is_extra, extra, sink))

        def stage(g):
            h = lax.rem(g, 2) * 8
            pltpu.sync_copy(src3.at[wid, pl.ds(g * 8, 8)], src_v.at[pl.ds(h, 8)])
            pltpu.sync_copy(dst3.at[wid, pl.ds(g * 8, 8)], dst_v.at[pl.ds(h, 8)])

        def gath(q, bs, bd, sem):
            k = lax.rem(q, 16)
            pltpu.async_copy(atom.at[src_v.at[k]], bs, sem)
            pltpu.async_copy(atom.at[dst_v.at[k]], bd, sem)

        def pair_wait(bs, bd, sem):
            pltpu.make_async_copy(atom.at[pl.ds(0, CG)], bs, sem).wait()
            pltpu.make_async_copy(atom.at[pl.ds(0, CG)], bd, sem).wait()

        def wr(q, bs, bd):
            pltpu.sync_copy(bd, sumh.at[pl.ds(ebase(q), CG)])
            pltpu.async_copy(bs, acc.at[dst_v.at[lax.rem(q, 16)]], swc, add=True)

        def scwait(buf):
            pltpu.make_async_copy(atom.at[pl.ds(0, CG)], buf, swc).wait()

        def comp(bs, bd):
            def row(r, carry):
                for j in range(D // LANES):
                    sl = pl.ds(j * LANES, LANES)
                    bd[r, sl] = bs[r, sl] + bd[r, sl]
                return carry

            lax.fori_loop(0, CG, row, 0)

        stage(0)
        gath(0, as0, ad0, sg0)

        def pair(t, carry):
            q0 = 2 * t
            q1 = q0 + 1

            @pl.when(t > 0)
            def _():
                scwait(as1)

            gath(q1, as1, ad1, sg1)
            pair_wait(as0, ad0, sg0)
            comp(as0, ad0)
            wr(q0, as0, ad0)
            pair_wait(as1, ad1, sg1)
            comp(as1, ad1)
            scwait(as0)

            @pl.when(jnp.logical_and(lax.rem(q0 + 2, 8) == 0, t + 1 < T))
            def _():
                stage((q0 + 2) // 8)

            @pl.when(t + 1 < T)
            def _():
                gath(q0 + 2, as0, ad0, sg0)

            wr(q1, as1, ad1)
            return carry

        lax.fori_loop(0, T, pair, 0)
        scwait(as1)
        plsc.subcore_barrier()
        pltpu.sync_copy(acc.at[pl.ds(s * NT, NT)],
                        part.at[c, pl.ds(s * NT, NT)])

    return pl.kernel(
        body,
        out_type=[jax.ShapeDtypeStruct((E + NW * CG, D), F32),
                  jax.ShapeDtypeStruct((NC, NP, D), F32)],
        mesh=_sc_mesh(),
        scratch_types=[
            pltpu.VMEM((16, CG), jnp.int32),
            pltpu.VMEM((16, CG), jnp.int32),
            pltpu.VMEM((CG, D), F32),
            pltpu.VMEM((CG, D), F32),
            pltpu.VMEM((CG, D), F32),
            pltpu.VMEM((CG, D), F32),
            pltpu.VMEM_SHARED((NP, D), F32),
            pltpu.SemaphoreType.DMA,
            pltpu.SemaphoreType.DMA,
            pltpu.SemaphoreType.DMA,
        ],
    )


def _make_segsum(N, E, D):
    EW = E // NW
    QN = EW // CH
    NP, NT = _pad_rows(N)

    T = -(-QN // 2)

    def body(evals, dst3, part, dst_v, e0, e1, acc, sr0, sr1, sw0, sw1):
        c = lax.axis_index("c")
        s = lax.axis_index("s")
        wid = c * NS + s
        _fill_zero(e0, CH, D)
        _zero_acc_rows(e0, CH, acc, s * NT, NT)
        plsc.subcore_barrier()

        def rd(q, buf, sem):
            pltpu.async_copy(evals.at[pl.ds(wid * EW + q * CH, CH)], buf, sem)

        def dma_wait(buf, sem):
            pltpu.make_async_copy(evals.at[pl.ds(0, CH)], buf, sem).wait()

        def sc(k, buf, sem):
            pltpu.async_copy(buf, acc.at[dst_v.at[k]], sem, add=True)

        rd(0, e0, sr0)

        def pair(t, carry):
            q0 = 2 * t
            q1 = q0 + 1

            @pl.when(t > 0)
            def _():
                dma_wait(e1, sw1)

            @pl.when(lax.rem(t, 4) == 0)
            def _():
                pltpu.sync_copy(dst3.at[wid, pl.ds((t // 4) * 8, 8)], dst_v)

            @pl.when(q1 < QN)
            def _():
                rd(q1, e1, sr1)

            dma_wait(e0, sr0)
            sc(lax.rem(q0, 8), e0, sw0)

            @pl.when(q1 < QN)
            def _():
                dma_wait(e1, sr1)

            dma_wait(e0, sw0)

            @pl.when(q0 + 2 < QN)
            def _():
                rd(q0 + 2, e0, sr0)

            @pl.when(q1 < QN)
            def _():
                sc(lax.rem(q1, 8), e1, sw1)

            return carry

        lax.fori_loop(0, T, pair, 0)
        if QN % 2 == 0:
            dma_wait(e1, sw1)
        plsc.subcore_barrier()
        pltpu.sync_copy(acc.at[pl.ds(s * NT, NT)],
                        part.at[c, pl.ds(s * NT, NT)])

    return pl.kernel(
        body,
        out_type=jax.ShapeDtypeStruct((NC, NP, D), F32),
        mesh=_sc_mesh(),
        scratch_types=[
            pltpu.VMEM((8, CH), jnp.int32),
            pltpu.VMEM((CH, D), F32),
            pltpu.VMEM((CH, D), F32),
            pltpu.VMEM_SHARED((NP, D), F32),
            pltpu.SemaphoreType.DMA,
            pltpu.SemaphoreType.DMA,
            pltpu.SemaphoreType.DMA,
            pltpu.SemaphoreType.DMA,
        ],
    )


def _make_edge_mlp(E, D, H):
    NB = E // BE

    def body(sumh_ref, bond_ref, w1a, w1b, b1, g, bt, w2, b2, e_ref,
             s1, s2, ca, cb):
        p = pl.program_id(0)
        j = pl.program_id(1)
        bf = jnp.bfloat16
        y = (jnp.dot(sumh_ref[...].astype(bf), w1a[...].astype(bf),
                     preferred_element_type=F32)
             + jnp.dot(bond_ref[...], w1b[...].astype(bf),
                       preferred_element_type=F32)
             + b1[...])

        @pl.when(p == 0)
        def _():
            @pl.when(j == 0)
            def _():
                s1[...] = jnp.zeros_like(s1)
                s2[...] = jnp.zeros_like(s2)

            s1[...] += jnp.sum(y, axis=0, keepdims=True)
            s2[...] += jnp.sum(y * y, axis=0, keepdims=True)

            @pl.when(j == NB - 1)
            def _():
                mu = s1[...] / E
                var = s2[...] / E - mu * mu
                a = g[...] * lax.rsqrt(var + EPS)
                ca[...] = a
                cb[...] = bt[...] - mu * a

        @pl.when(p == 1)
        def _():
            yh = jnp.maximum(y * ca[...] + cb[...], 0.0)
            e_ref[...] = jnp.dot(yh.astype(bf), w2[...].astype(bf),
                                 preferred_element_type=F32) + b2[...]

    return pl.pallas_call(
        body,
        grid=(2, NB),
        in_specs=[
            pl.BlockSpec((BE, D), lambda p, j: (j, 0)),
            pl.BlockSpec((BE, D), lambda p, j: (j, 0)),
            pl.BlockSpec((D, H), lambda p, j: (0, 0)),
            pl.BlockSpec((D, H), lambda p, j: (0, 0)),
            pl.BlockSpec((1, H), lambda p, j: (0, 0)),
            pl.BlockSpec((1, H), lambda p, j: (0, 0)),
            pl.BlockSpec((1, H), lambda p, j: (0, 0)),
            pl.BlockSpec((H, D), lambda p, j: (0, 0)),
            pl.BlockSpec((1, D), lambda p, j: (0, 0)),
        ],
        out_specs=pl.BlockSpec((BE, D), lambda p, j: (jnp.where(p == 0, 0, j), 0)),
        out_shape=jax.ShapeDtypeStruct((E, D), F32),
        scratch_shapes=[pltpu.VMEM((1, H), F32)] * 4,
        compiler_params=pltpu.CompilerParams(
            dimension_semantics=("arbitrary", "arbitrary")),
    )


def _make_node_mlp(N, D, H):
    def body(ph, pe, w1a, w1b, b1, g, bt, w2, b2, h_ref):
        bf = jnp.bfloat16
        sh = ph[0, :N] + ph[1, :N]
        se = pe[0, :N] + pe[1, :N]
        y = (jnp.dot(sh.astype(bf), w1a[...].astype(bf),
                     preferred_element_type=F32)
             + jnp.dot(se.astype(bf), w1b[...].astype(bf),
                       preferred_element_type=F32)
             + b1[...])
        mu = jnp.mean(y, axis=0, keepdims=True)
        var = jnp.mean(y * y, axis=0, keepdims=True) - mu * mu
        yh = jnp.maximum((y - mu) * (g[...] * lax.rsqrt(var + EPS)) + bt[...], 0.0)
        h_ref[...] = jnp.dot(yh.astype(bf), w2[...].astype(bf),
                             preferred_element_type=F32) + b2[...]

    return pl.pallas_call(
        body,
        out_shape=jax.ShapeDtypeStruct((N, D), F32),
    )


def kernel(atom, bond, edge_index, Wa1, ba1, ga, bta, Wa2, ba2,
           Wb1, bb1, gb, btb, Wb2, bb2):
    N, D = atom.shape
    E = bond.shape[0]
    H = Wb1.shape[1]
    assert E % (NW * CH) == 0 and E % BE == 0 and E % CG == 0
    QN = E // (NW * CH)
    pad = -(-QN // 8) * 8 - QN

    dst3 = jnp.pad(edge_index[1].reshape(NW, QN, CH), ((0, 0), (0, pad), (0, 0)))

    _, QMAIN, XTRA, _, NQPAD = _gather_geometry(N, E)
    nmain = QMAIN * NW * CG
    src_m = edge_index[0, :nmain].reshape(NW, QMAIN, CG)
    dst_m = edge_index[1, :nmain].reshape(NW, QMAIN, CG)
    tail_src = jnp.zeros((NW, NQPAD - QMAIN, CG), jnp.int32)
    tail_dst = jnp.full((NW, NQPAD - QMAIN, CG), N, jnp.int32)
    if XTRA:
        tail_src = tail_src.at[:XTRA, 0].set(
            edge_index[0, nmain:].reshape(XTRA, CG))
        tail_dst = tail_dst.at[:XTRA, 0].set(
            edge_index[1, nmain:].reshape(XTRA, CG))
    src4 = jnp.concatenate([src_m, tail_src], axis=1)
    dst4 = jnp.concatenate([dst_m, tail_dst], axis=1)

    bond16 = bond.astype(jnp.bfloat16)
    sumh, ph = _make_gather_segsum(N, E, D)(atom, src4, dst4)
    e = _make_edge_mlp(E, D, H)(
        sumh, bond16, Wb1[:D], Wb1[D:], bb1.reshape(1, H), gb.reshape(1, H),
        btb.reshape(1, H), Wb2, bb2.reshape(1, D))
    pe = _make_segsum(N, E, D)(e, dst3)
    h = _make_node_mlp(N, D, H)(
        ph, pe, Wa1[:D], Wa1[D:], ba1.reshape(1, H), ga.reshape(1, H),
        bta.reshape(1, H), Wa2, ba2.reshape(1, D))
    return h, e


# speedup vs baseline: 1.0196x; 1.0196x over previous
import jax
import jax.numpy as jnp
from jax import lax
from jax.experimental import pallas as pl
from jax.experimental.pallas import tpu as pltpu
from jax.experimental.pallas import tpu_sc as plsc

F32 = jnp.float32
EPS = 1e-5
NC = 2
NS = 16
NW = NC * NS
LANES = 16
CH = 80
BE = 2560


def _sc_mesh():
    return plsc.VectorSubcoreMesh(
        core_axis_name="c", subcore_axis_name="s",
        num_cores=NC, num_subcores=NS)


def _fill_zero(buf, rows, cols):
    zero = jnp.zeros((LANES,), F32)

    def body(r, carry):
        for j in range(cols // LANES):
            buf[r, pl.ds(j * LANES, LANES)] = zero
        return carry

    lax.fori_loop(0, rows, body, 0)


def _zero_acc_rows(src_buf, rows, acc, base, nrows):
    nfull, rem = divmod(nrows, rows)
    for k in range(nfull):
        pltpu.sync_copy(src_buf, acc.at[pl.ds(base + k * rows, rows)])
    if rem:
        pltpu.sync_copy(src_buf.at[pl.ds(0, rem)],
                        acc.at[pl.ds(base + nfull * rows, rem)])


def _pad_rows(N):
    nt = -(-N // NS)
    nt = -(-nt // 8) * 8
    return nt * NS, nt


CG = 64


def _gather_geometry(N, E):
    QTOT = E // CG
    QMAIN = QTOT // NW
    XTRA = QTOT - QMAIN * NW
    NQ = QMAIN + 1
    NQ += NQ % 2
    NQPAD = -(-NQ // 8) * 8
    return QTOT, QMAIN, XTRA, NQ, NQPAD


def _make_gather_segsum(N, E, D):
    QTOT, QMAIN, XTRA, NQ, NQPAD = _gather_geometry(N, E)
    NP, NT = _pad_rows(N)
    T = NQ // 2
    MW = QMAIN * CG

    def body(atom, src3, dst3, sumh, part, src_v, dst_v,
             as0, ad0, as1, ad1, acc, sg0, sg1, swc):
        c = lax.axis_index("c")
        s = lax.axis_index("s")
        wid = c * NS + s
        _fill_zero(ad0, CG, D)
        _zero_acc_rows(ad0, CG, acc, s * NT, NT)
        plsc.subcore_barrier()

        def ebase(q):
            main = wid * MW + q * CG
            extra = QMAIN * NW * CG + wid * CG
            sink = E + wid * CG
            is_extra = jnp.logical_and(q == QMAIN, wid < XTRA)
            return jnp.where(q < QMAIN, main, jnp.where(is_extra, extra, sink))

        def stage(g):
            h = lax.rem(g, 2) * 8
            pltpu.sync_copy(src3.at[wid, pl.ds(g * 8, 8)], src_v.at[pl.ds(h, 8)])
            pltpu.sync_copy(dst3.at[wid, pl.ds(g * 8, 8)], dst_v.at[pl.ds(h, 8)])

        def gath(q, bs, bd, sem):
            k = lax.rem(q, 16)
            pltpu.async_copy(atom.at[src_v.at[k]], bs, sem)
            pltpu.async_copy(atom.at[dst_v.at[k]], bd, sem)

        def pair_wait(bs, bd, sem):
            pltpu.make_async_copy(atom.at[pl.ds(0, CG)], bs, sem).wait()
            pltpu.make_async_copy(atom.at[pl.ds(0, CG)], bd, sem).wait()

        def wr(q, bs, bd):
            pltpu.sync_copy(bd, sumh.at[pl.ds(ebase(q), CG)])
            pltpu.async_copy(bs, acc.at[dst_v.at[lax.rem(q, 16)]], swc, add=True)

        def scwait(buf):
            pltpu.make_async_copy(atom.at[pl.ds(0, CG)], buf, swc).wait()

        def comp(bs, bd):
            def row(r, carry):
                for j in range(D // LANES):
                    sl = pl.ds(j * LANES, LANES)
                    bd[r, sl] = bs[r, sl] + bd[r, sl]
                return carry

            lax.fori_loop(0, CG, row, 0)

        stage(0)
        gath(0, as0, ad0, sg0)

        def pair(t, carry):
            q0 = 2 * t
            q1 = q0 + 1

            @pl.when(t > 0)
            def _():
                scwait(as1)

            gath(q1, as1, ad1, sg1)
            pair_wait(as0, ad0, sg0)
            comp(as0, ad0)
            wr(q0, as0, ad0)
            pair_wait(as1, ad1, sg1)
            comp(as1, ad1)
            scwait(as0)

            @pl.when(jnp.logical_and(lax.rem(q0 + 2, 8) == 0, t + 1 < T))
            def _():
                stage((q0 + 2) // 8)

            @pl.when(t + 1 < T)
            def _():
                gath(q0 + 2, as0, ad0, sg0)

            wr(q1, as1, ad1)
            return carry

        lax.fori_loop(0, T, pair, 0)
        scwait(as1)
        plsc.subcore_barrier()
        pltpu.sync_copy(acc.at[pl.ds(s * NT, NT)],
                        part.at[c, pl.ds(s * NT, NT)])

    return pl.kernel(
        body,
        out_type=[jax.ShapeDtypeStruct((E + NW * CG, D), F32),
                  jax.ShapeDtypeStruct((NC, NP, D), F32)],
        mesh=_sc_mesh(),
        scratch_types=[
            pltpu.VMEM((16, CG), jnp.int32),
            pltpu.VMEM((16, CG), jnp.int32),
            pltpu.VMEM((CG, D), F32),
            pltpu.VMEM((CG, D), F32),
            pltpu.VMEM((CG, D), F32),
            pltpu.VMEM((CG, D), F32),
            pltpu.VMEM_SHARED((NP, D), F32),
            pltpu.SemaphoreType.DMA,
            pltpu.SemaphoreType.DMA,
            pltpu.SemaphoreType.DMA,
        ],
    )


def _make_segsum(N, E, D):
    EW = E // NW
    QN = EW // CH
    NP, NT = _pad_rows(N)

    T = -(-QN // 2)

    def body(evals, dst3, part, dst_v, e0, e1, acc, sr0, sr1, sw0, sw1):
        c = lax.axis_index("c")
        s = lax.axis_index("s")
        wid = c * NS + s
        _fill_zero(e0, CH, D)
        _zero_acc_rows(e0, CH, acc, s * NT, NT)
        plsc.subcore_barrier()

        def rd(q, buf, sem):
            pltpu.async_copy(evals.at[pl.ds(wid * EW + q * CH, CH)], buf, sem)

        def dma_wait(buf, sem):
            pltpu.make_async_copy(evals.at[pl.ds(0, CH)], buf, sem).wait()

        def sc(k, buf, sem):
            pltpu.async_copy(buf, acc.at[dst_v.at[k]], sem, add=True)

        rd(0, e0, sr0)

        def pair(t, carry):
            q0 = 2 * t
            q1 = q0 + 1

            @pl.when(t > 0)
            def _():
                dma_wait(e1, sw1)

            @pl.when(lax.rem(t, 4) == 0)
            def _():
                pltpu.sync_copy(dst3.at[wid, pl.ds((t // 4) * 8, 8)], dst_v)

            @pl.when(q1 < QN)
            def _():
                rd(q1, e1, sr1)

            dma_wait(e0, sr0)
            sc(lax.rem(q0, 8), e0, sw0)

            @pl.when(q1 < QN)
            def _():
                dma_wait(e1, sr1)

            dma_wait(e0, sw0)

            @pl.when(q0 + 2 < QN)
            def _():
                rd(q0 + 2, e0, sr0)

            @pl.when(q1 < QN)
            def _():
                sc(lax.rem(q1, 8), e1, sw1)

            return carry

        lax.fori_loop(0, T, pair, 0)
        if QN % 2 == 0:
            dma_wait(e1, sw1)
        plsc.subcore_barrier()
        pltpu.sync_copy(acc.at[pl.ds(s * NT, NT)],
                        part.at[c, pl.ds(s * NT, NT)])

    return pl.kernel(
        body,
        out_type=jax.ShapeDtypeStruct((NC, NP, D), F32),
        mesh=_sc_mesh(),
        scratch_types=[
            pltpu.VMEM((8, CH), jnp.int32),
            pltpu.VMEM((CH, D), F32),
            pltpu.VMEM((CH, D), F32),
            pltpu.VMEM_SHARED((NP, D), F32),
            pltpu.SemaphoreType.DMA,
            pltpu.SemaphoreType.DMA,
            pltpu.SemaphoreType.DMA,
            pltpu.SemaphoreType.DMA,
        ],
    )


def _make_edge_mlp(E, D, H):
    NB = E // BE

    def body(sumh_ref, bond_ref, w1a, w1b, b1, g, bt, w2, b2, e_ref,
             s1, s2, ca, cb):
        p = pl.program_id(0)
        j = pl.program_id(1)
        bf = jnp.bfloat16
        y = (jnp.dot(sumh_ref[...].astype(bf), w1a[...].astype(bf),
                     preferred_element_type=F32)
             + jnp.dot(bond_ref[...].astype(bf), w1b[...].astype(bf),
                       preferred_element_type=F32)
             + b1[...])

        @pl.when(p == 0)
        def _():
            @pl.when(j == 0)
            def _():
                s1[...] = jnp.zeros_like(s1)
                s2[...] = jnp.zeros_like(s2)

            s1[...] += jnp.sum(y, axis=0, keepdims=True)
            s2[...] += jnp.sum(y * y, axis=0, keepdims=True)

            @pl.when(j == NB - 1)
            def _():
                mu = s1[...] / E
                var = s2[...] / E - mu * mu
                a = g[...] * lax.rsqrt(var + EPS)
                ca[...] = a
                cb[...] = bt[...] - mu * a

        @pl.when(p == 1)
        def _():
            yh = jnp.maximum(y * ca[...] + cb[...], 0.0)
            e_ref[...] = jnp.dot(yh.astype(bf), w2[...].astype(bf),
                                 preferred_element_type=F32) + b2[...]

    return pl.pallas_call(
        body,
        grid=(2, NB),
        in_specs=[
            pl.BlockSpec((BE, D), lambda p, j: (j, 0)),
            pl.BlockSpec((BE, D), lambda p, j: (j, 0)),
            pl.BlockSpec((D, H), lambda p, j: (0, 0)),
            pl.BlockSpec((D, H), lambda p, j: (0, 0)),
            pl.BlockSpec((1, H), lambda p, j: (0, 0)),
            pl.BlockSpec((1, H), lambda p, j: (0, 0)),
            pl.BlockSpec((1, H), lambda p, j: (0, 0)),
            pl.BlockSpec((H, D), lambda p, j: (0, 0)),
            pl.BlockSpec((1, D), lambda p, j: (0, 0)),
        ],
        out_specs=pl.BlockSpec((BE, D), lambda p, j: (jnp.where(p == 0, 0, j), 0)),
        out_shape=jax.ShapeDtypeStruct((E, D), F32),
        scratch_shapes=[pltpu.VMEM((1, H), F32)] * 4,
        compiler_params=pltpu.CompilerParams(
            dimension_semantics=("arbitrary", "arbitrary")),
    )


def _make_node_mlp(N, D, H):
    def body(ph, pe, w1a, w1b, b1, g, bt, w2, b2, h_ref):
        bf = jnp.bfloat16
        sh = ph[0, :N] + ph[1, :N]
        se = pe[0, :N] + pe[1, :N]
        y = (jnp.dot(sh.astype(bf), w1a[...].astype(bf),
                     preferred_element_type=F32)
             + jnp.dot(se.astype(bf), w1b[...].astype(bf),
                       preferred_element_type=F32)
             + b1[...])
        mu = jnp.mean(y, axis=0, keepdims=True)
        var = jnp.mean(y * y, axis=0, keepdims=True) - mu * mu
        yh = jnp.maximum((y - mu) * (g[...] * lax.rsqrt(var + EPS)) + bt[...], 0.0)
        h_ref[...] = jnp.dot(yh.astype(bf), w2[...].astype(bf),
                             preferred_element_type=F32) + b2[...]

    return pl.pallas_call(
        body,
        out_shape=jax.ShapeDtypeStruct((N, D), F32),
    )


def kernel(atom, bond, edge_index, Wa1, ba1, ga, bta, Wa2, ba2,
           Wb1, bb1, gb, btb, Wb2, bb2):
    N, D = atom.shape
    E = bond.shape[0]
    H = Wb1.shape[1]
    assert E % (NW * CH) == 0 and E % BE == 0 and E % CG == 0
    QN = E // (NW * CH)
    pad = -(-QN // 8) * 8 - QN

    dst3 = jnp.pad(edge_index[1].reshape(NW, QN, CH), ((0, 0), (0, pad), (0, 0)))

    _, QMAIN, XTRA, _, NQPAD = _gather_geometry(N, E)
    nmain = QMAIN * NW * CG
    src_m = edge_index[0, :nmain].reshape(NW, QMAIN, CG)
    dst_m = edge_index[1, :nmain].reshape(NW, QMAIN, CG)
    tail_src = jnp.zeros((NW, NQPAD - QMAIN, CG), jnp.int32)
    tail_dst = jnp.full((NW, NQPAD - QMAIN, CG), N, jnp.int32)
    if XTRA:
        tail_src = tail_src.at[:XTRA, 0].set(
            edge_index[0, nmain:].reshape(XTRA, CG))
        tail_dst = tail_dst.at[:XTRA, 0].set(
            edge_index[1, nmain:].reshape(XTRA, CG))
    src4 = jnp.concatenate([src_m, tail_src], axis=1)
    dst4 = jnp.concatenate([dst_m, tail_dst], axis=1)

    sumh, ph = _make_gather_segsum(N, E, D)(atom, src4, dst4)
    e = _make_edge_mlp(E, D, H)(
        sumh, bond, Wb1[:D], Wb1[D:], bb1.reshape(1, H), gb.reshape(1, H),
        btb.reshape(1, H), Wb2, bb2.reshape(1, D))
    pe = _make_segsum(N, E, D)(e, dst3)
    h = _make_node_mlp(N, D, H)(
        ph, pe, Wa1[:D], Wa1[D:], ba1.reshape(1, H), ga.reshape(1, H),
        bta.reshape(1, H), Wa2, ba2.reshape(1, D))
    return h, e


# speedup vs baseline: 1.0838x; 1.0629x over previous
import jax
import jax.numpy as jnp
from jax import lax
from jax.experimental import pallas as pl
from jax.experimental.pallas import tpu as pltpu
from jax.experimental.pallas import tpu_sc as plsc

F32 = jnp.float32
EPS = 1e-5
NC = 2
NS = 16
NW = NC * NS
LANES = 16
CH = 80
BE = 4000


def _sc_mesh():
    return plsc.VectorSubcoreMesh(
        core_axis_name="c", subcore_axis_name="s",
        num_cores=NC, num_subcores=NS)


def _fill_zero(buf, rows, cols):
    zero = jnp.zeros((LANES,), F32)

    def body(r, carry):
        for j in range(cols // LANES):
            buf[r, pl.ds(j * LANES, LANES)] = zero
        return carry

    lax.fori_loop(0, rows, body, 0)


def _zero_acc_rows(src_buf, rows, acc, base, nrows):
    nfull, rem = divmod(nrows, rows)
    for k in range(nfull):
        pltpu.sync_copy(src_buf, acc.at[pl.ds(base + k * rows, rows)])
    if rem:
        pltpu.sync_copy(src_buf.at[pl.ds(0, rem)],
                        acc.at[pl.ds(base + nfull * rows, rem)])


def _pad_rows(N):
    nt = -(-N // NS)
    nt = -(-nt // 8) * 8
    return nt * NS, nt


CG = 64


def _gather_geometry(N, E):
    QTOT = E // CG
    QMAIN = QTOT // NW
    XTRA = QTOT - QMAIN * NW
    NQ = QMAIN + 1
    NQ += NQ % 2
    NQPAD = -(-NQ // 8) * 8
    return QTOT, QMAIN, XTRA, NQ, NQPAD


def _make_gather_segsum(N, E, D):
    QTOT, QMAIN, XTRA, NQ, NQPAD = _gather_geometry(N, E)
    NP, NT = _pad_rows(N)
    T = NQ // 2
    MW = QMAIN * CG

    def body(atom, src3, dst3, sumh, part, src_v, dst_v,
             as0, ad0, as1, ad1, acc, sg0, sg1, swc):
        c = lax.axis_index("c")
        s = lax.axis_index("s")
        wid = c * NS + s
        _fill_zero(ad0, CG, D)
        _zero_acc_rows(ad0, CG, acc, s * NT, NT)
        plsc.subcore_barrier()

        def ebase(q):
            main = wid * MW + q * CG
            extra = QMAIN * NW * CG + wid * CG
            sink = E + wid * CG
            is_extra = jnp.logical_and(q == QMAIN, wid < XTRA)
            return jnp.where(q < QMAIN, main, jnp.where(is_extra, extra, sink))

        def stage(g):
            h = lax.rem(g, 2) * 8
            pltpu.sync_copy(src3.at[wid, pl.ds(g * 8, 8)], src_v.at[pl.ds(h, 8)])
            pltpu.sync_copy(dst3.at[wid, pl.ds(g * 8, 8)], dst_v.at[pl.ds(h, 8)])

        def gath(q, bs, bd, sem):
            k = lax.rem(q, 16)
            pltpu.async_copy(atom.at[src_v.at[k]], bs, sem)
            pltpu.async_copy(atom.at[dst_v.at[k]], bd, sem)

        def pair_wait(bs, bd, sem):
            pltpu.make_async_copy(atom.at[pl.ds(0, CG)], bs, sem).wait()
            pltpu.make_async_copy(atom.at[pl.ds(0, CG)], bd, sem).wait()

        def wr(q, bs, bd):
            pltpu.sync_copy(bd, sumh.at[pl.ds(ebase(q), CG)])
            pltpu.async_copy(bs, acc.at[dst_v.at[lax.rem(q, 16)]], swc, add=True)

        def scwait(buf):
            pltpu.make_async_copy(atom.at[pl.ds(0, CG)], buf, swc).wait()

        def comp(bs, bd):
            def row(r, carry):
                for j in range(D // LANES):
                    sl = pl.ds(j * LANES, LANES)
                    bd[r, sl] = bs[r, sl] + bd[r, sl]
                return carry

            lax.fori_loop(0, CG, row, 0)

        stage(0)
        gath(0, as0, ad0, sg0)

        def pair(t, carry):
            q0 = 2 * t
            q1 = q0 + 1

            @pl.when(t > 0)
            def _():
                scwait(as1)

            gath(q1, as1, ad1, sg1)
            pair_wait(as0, ad0, sg0)
            comp(as0, ad0)
            wr(q0, as0, ad0)
            pair_wait(as1, ad1, sg1)
            comp(as1, ad1)
            scwait(as0)

            @pl.when(jnp.logical_and(lax.rem(q0 + 2, 8) == 0, t + 1 < T))
            def _():
                stage((q0 + 2) // 8)

            @pl.when(t + 1 < T)
            def _():
                gath(q0 + 2, as0, ad0, sg0)

            wr(q1, as1, ad1)
            return carry

        lax.fori_loop(0, T, pair, 0)
        scwait(as1)
        plsc.subcore_barrier()
        pltpu.sync_copy(acc.at[pl.ds(s * NT, NT)],
                        part.at[c, pl.ds(s * NT, NT)])

    return pl.kernel(
        body,
        out_type=[jax.ShapeDtypeStruct((E + NW * CG, D), F32),
                  jax.ShapeDtypeStruct((NC, NP, D), F32)],
        mesh=_sc_mesh(),
        scratch_types=[
            pltpu.VMEM((16, CG), jnp.int32),
            pltpu.VMEM((16, CG), jnp.int32),
            pltpu.VMEM((CG, D), F32),
            pltpu.VMEM((CG, D), F32),
            pltpu.VMEM((CG, D), F32),
            pltpu.VMEM((CG, D), F32),
            pltpu.VMEM_SHARED((NP, D), F32),
            pltpu.SemaphoreType.DMA,
            pltpu.SemaphoreType.DMA,
            pltpu.SemaphoreType.DMA,
        ],
    )


def _make_segsum(N, E, D):
    EW = E // NW
    QN = EW // CH
    NP, NT = _pad_rows(N)

    T = -(-QN // 2)

    def body(evals, dst3, part, dst_v, e0, e1, acc, sr0, sr1, sw0, sw1):
        c = lax.axis_index("c")
        s = lax.axis_index("s")
        wid = c * NS + s
        _fill_zero(e0, CH, D)
        _zero_acc_rows(e0, CH, acc, s * NT, NT)
        plsc.subcore_barrier()

        def rd(q, buf, sem):
            pltpu.async_copy(evals.at[pl.ds(wid * EW + q * CH, CH)], buf, sem)

        def dma_wait(buf, sem):
            pltpu.make_async_copy(evals.at[pl.ds(0, CH)], buf, sem).wait()

        def sc(k, buf, sem):
            pltpu.async_copy(buf, acc.at[dst_v.at[k]], sem, add=True)

        rd(0, e0, sr0)

        def pair(t, carry):
            q0 = 2 * t
            q1 = q0 + 1

            @pl.when(t > 0)
            def _():
                dma_wait(e1, sw1)

            @pl.when(lax.rem(t, 4) == 0)
            def _():
                pltpu.sync_copy(dst3.at[wid, pl.ds((t // 4) * 8, 8)], dst_v)

            @pl.when(q1 < QN)
            def _():
                rd(q1, e1, sr1)

            dma_wait(e0, sr0)
            sc(lax.rem(q0, 8), e0, sw0)

            @pl.when(q1 < QN)
            def _():
                dma_wait(e1, sr1)

            dma_wait(e0, sw0)

            @pl.when(q0 + 2 < QN)
            def _():
                rd(q0 + 2, e0, sr0)

            @pl.when(q1 < QN)
            def _():
                sc(lax.rem(q1, 8), e1, sw1)

            return carry

        lax.fori_loop(0, T, pair, 0)
        if QN % 2 == 0:
            dma_wait(e1, sw1)
        plsc.subcore_barrier()
        pltpu.sync_copy(acc.at[pl.ds(s * NT, NT)],
                        part.at[c, pl.ds(s * NT, NT)])

    return pl.kernel(
        body,
        out_type=jax.ShapeDtypeStruct((NC, NP, D), F32),
        mesh=_sc_mesh(),
        scratch_types=[
            pltpu.VMEM((8, CH), jnp.int32),
            pltpu.VMEM((CH, D), F32),
            pltpu.VMEM((CH, D), F32),
            pltpu.VMEM_SHARED((NP, D), F32),
            pltpu.SemaphoreType.DMA,
            pltpu.SemaphoreType.DMA,
            pltpu.SemaphoreType.DMA,
            pltpu.SemaphoreType.DMA,
        ],
    )


def _make_edge_mlp(E, D, H):
    NB = E // BE

    def body(sumh_ref, bond_ref, w1a, w1b, b1, g, bt, w2, b2, e_ref,
             s1, s2, ca, cb):
        p = pl.program_id(0)
        j = pl.program_id(1)
        bf = jnp.bfloat16
        y = (jnp.dot(sumh_ref[...].astype(bf), w1a[...].astype(bf),
                     preferred_element_type=F32)
             + jnp.dot(bond_ref[...].astype(bf), w1b[...].astype(bf),
                       preferred_element_type=F32)
             + b1[...])

        @pl.when(p == 0)
        def _():
            @pl.when(j == 0)
            def _():
                s1[...] = jnp.zeros_like(s1)
                s2[...] = jnp.zeros_like(s2)

            s1[...] += jnp.sum(y, axis=0, keepdims=True)
            s2[...] += jnp.sum(y * y, axis=0, keepdims=True)

            @pl.when(j == NB - 1)
            def _():
                mu = s1[...] / E
                var = s2[...] / E - mu * mu
                a = g[...] * lax.rsqrt(var + EPS)
                ca[...] = a
                cb[...] = bt[...] - mu * a

        @pl.when(p == 1)
        def _():
            yh = jnp.maximum(y * ca[...] + cb[...], 0.0)
            e_ref[...] = jnp.dot(yh.astype(bf), w2[...].astype(bf),
                                 preferred_element_type=F32) + b2[...]

    return pl.pallas_call(
        body,
        grid=(2, NB),
        in_specs=[
            pl.BlockSpec((BE, D), lambda p, j: (j, 0)),
            pl.BlockSpec((BE, D), lambda p, j: (j, 0)),
            pl.BlockSpec((D, H), lambda p, j: (0, 0)),
            pl.BlockSpec((D, H), lambda p, j: (0, 0)),
            pl.BlockSpec((1, H), lambda p, j: (0, 0)),
            pl.BlockSpec((1, H), lambda p, j: (0, 0)),
            pl.BlockSpec((1, H), lambda p, j: (0, 0)),
            pl.BlockSpec((H, D), lambda p, j: (0, 0)),
            pl.BlockSpec((1, D), lambda p, j: (0, 0)),
        ],
        out_specs=pl.BlockSpec((BE, D), lambda p, j: (jnp.where(p == 0, 0, j), 0)),
        out_shape=jax.ShapeDtypeStruct((E, D), F32),
        scratch_shapes=[pltpu.VMEM((1, H), F32)] * 4,
        compiler_params=pltpu.CompilerParams(
            dimension_semantics=("arbitrary", "arbitrary")),
    )


def _make_node_mlp(N, D, H):
    def body(ph, pe, w1a, w1b, b1, g, bt, w2, b2, h_ref):
        bf = jnp.bfloat16
        sh = ph[0, :N] + ph[1, :N]
        se = pe[0, :N] + pe[1, :N]
        y = (jnp.dot(sh.astype(bf), w1a[...].astype(bf),
                     preferred_element_type=F32)
             + jnp.dot(se.astype(bf), w1b[...].astype(bf),
                       preferred_element_type=F32)
             + b1[...])
        mu = jnp.mean(y, axis=0, keepdims=True)
        var = jnp.mean(y * y, axis=0, keepdims=True) - mu * mu
        yh = jnp.maximum((y - mu) * (g[...] * lax.rsqrt(var + EPS)) + bt[...], 0.0)
        h_ref[...] = jnp.dot(yh.astype(bf), w2[...].astype(bf),
                             preferred_element_type=F32) + b2[...]

    return pl.pallas_call(
        body,
        out_shape=jax.ShapeDtypeStruct((N, D), F32),
    )


def kernel(atom, bond, edge_index, Wa1, ba1, ga, bta, Wa2, ba2,
           Wb1, bb1, gb, btb, Wb2, bb2):
    N, D = atom.shape
    E = bond.shape[0]
    H = Wb1.shape[1]
    assert E % (NW * CH) == 0 and E % BE == 0 and E % CG == 0
    QN = E // (NW * CH)
    pad = -(-QN // 8) * 8 - QN

    dst3 = jnp.pad(edge_index[1].reshape(NW, QN, CH), ((0, 0), (0, pad), (0, 0)))

    _, QMAIN, XTRA, _, NQPAD = _gather_geometry(N, E)
    nmain = QMAIN * NW * CG
    src_m = edge_index[0, :nmain].reshape(NW, QMAIN, CG)
    dst_m = edge_index[1, :nmain].reshape(NW, QMAIN, CG)
    tail_src = jnp.zeros((NW, NQPAD - QMAIN, CG), jnp.int32)
    tail_dst = jnp.full((NW, NQPAD - QMAIN, CG), N, jnp.int32)
    if XTRA:
        tail_src = tail_src.at[:XTRA, 0].set(
            edge_index[0, nmain:].reshape(XTRA, CG))
        tail_dst = tail_dst.at[:XTRA, 0].set(
            edge_index[1, nmain:].reshape(XTRA, CG))
    src4 = jnp.concatenate([src_m, tail_src], axis=1)
    dst4 = jnp.concatenate([dst_m, tail_dst], axis=1)

    sumh, ph = _make_gather_segsum(N, E, D)(atom, src4, dst4)
    e = _make_edge_mlp(E, D, H)(
        sumh, bond, Wb1[:D], Wb1[D:], bb1.reshape(1, H), gb.reshape(1, H),
        btb.reshape(1, H), Wb2, bb2.reshape(1, D))
    pe = _make_segsum(N, E, D)(e, dst3)
    h = _make_node_mlp(N, D, H)(
        ph, pe, Wa1[:D], Wa1[D:], ba1.reshape(1, H), ga.reshape(1, H),
        bta.reshape(1, H), Wa2, ba2.reshape(1, D))
    return h, e


# speedup vs baseline: 1.1463x; 1.0577x over previous
import jax
import jax.numpy as jnp
from jax import lax
from jax.experimental import pallas as pl
from jax.experimental.pallas import tpu as pltpu
from jax.experimental.pallas import tpu_sc as plsc

F32 = jnp.float32
EPS = 1e-5
NC = 2
NS = 16
NW = NC * NS
LANES = 16
CH = 80
BE = 8000


def _sc_mesh():
    return plsc.VectorSubcoreMesh(
        core_axis_name="c", subcore_axis_name="s",
        num_cores=NC, num_subcores=NS)


def _fill_zero(buf, rows, cols):
    zero = jnp.zeros((LANES,), F32)

    def body(r, carry):
        for j in range(cols // LANES):
            buf[r, pl.ds(j * LANES, LANES)] = zero
        return carry

    lax.fori_loop(0, rows, body, 0)


def _zero_acc_rows(src_buf, rows, acc, base, nrows):
    nfull, rem = divmod(nrows, rows)
    for k in range(nfull):
        pltpu.sync_copy(src_buf, acc.at[pl.ds(base + k * rows, rows)])
    if rem:
        pltpu.sync_copy(src_buf.at[pl.ds(0, rem)],
                        acc.at[pl.ds(base + nfull * rows, rem)])


def _pad_rows(N):
    nt = -(-N // NS)
    nt = -(-nt // 8) * 8
    return nt * NS, nt


CG = 64


def _gather_geometry(N, E):
    QTOT = E // CG
    QMAIN = QTOT // NW
    XTRA = QTOT - QMAIN * NW
    NQ = QMAIN + 1
    NQ += NQ % 2
    NQPAD = -(-NQ // 8) * 8
    return QTOT, QMAIN, XTRA, NQ, NQPAD


def _make_gather_segsum(N, E, D):
    QTOT, QMAIN, XTRA, NQ, NQPAD = _gather_geometry(N, E)
    NP, NT = _pad_rows(N)
    T = NQ // 2
    MW = QMAIN * CG

    def body(atom, src3, dst3, sumh, part, src_v, dst_v,
             as0, ad0, as1, ad1, acc, sg0, sg1, swc):
        c = lax.axis_index("c")
        s = lax.axis_index("s")
        wid = c * NS + s
        _fill_zero(ad0, CG, D)
        _zero_acc_rows(ad0, CG, acc, s * NT, NT)
        plsc.subcore_barrier()

        def ebase(q):
            main = wid * MW + q * CG
            extra = QMAIN * NW * CG + wid * CG
            sink = E + wid * CG
            is_extra = jnp.logical_and(q == QMAIN, wid < XTRA)
            return jnp.where(q < QMAIN, main, jnp.where(is_extra, extra, sink))

        def stage(g):
            h = lax.rem(g, 2) * 8
            pltpu.sync_copy(src3.at[wid, pl.ds(g * 8, 8)], src_v.at[pl.ds(h, 8)])
            pltpu.sync_copy(dst3.at[wid, pl.ds(g * 8, 8)], dst_v.at[pl.ds(h, 8)])

        def gath(q, bs, bd, sem):
            k = lax.rem(q, 16)
            pltpu.async_copy(atom.at[src_v.at[k]], bs, sem)
            pltpu.async_copy(atom.at[dst_v.at[k]], bd, sem)

        def pair_wait(bs, bd, sem):
            pltpu.make_async_copy(atom.at[pl.ds(0, CG)], bs, sem).wait()
            pltpu.make_async_copy(atom.at[pl.ds(0, CG)], bd, sem).wait()

        def wr(q, bs, bd):
            pltpu.sync_copy(bd, sumh.at[pl.ds(ebase(q), CG)])
            pltpu.async_copy(bs, acc.at[dst_v.at[lax.rem(q, 16)]], swc, add=True)

        def scwait(buf):
            pltpu.make_async_copy(atom.at[pl.ds(0, CG)], buf, swc).wait()

        def comp(bs, bd):
            def row(r, carry):
                for j in range(D // LANES):
                    sl = pl.ds(j * LANES, LANES)
                    bd[r, sl] = bs[r, sl] + bd[r, sl]
                return carry

            lax.fori_loop(0, CG, row, 0)

        stage(0)
        gath(0, as0, ad0, sg0)

        def pair(t, carry):
            q0 = 2 * t
            q1 = q0 + 1

            @pl.when(t > 0)
            def _():
                scwait(as1)

            gath(q1, as1, ad1, sg1)
            pair_wait(as0, ad0, sg0)
            comp(as0, ad0)
            wr(q0, as0, ad0)
            pair_wait(as1, ad1, sg1)
            comp(as1, ad1)
            scwait(as0)

            @pl.when(jnp.logical_and(lax.rem(q0 + 2, 8) == 0, t + 1 < T))
            def _():
                stage((q0 + 2) // 8)

            @pl.when(t + 1 < T)
            def _():
                gath(q0 + 2, as0, ad0, sg0)

            wr(q1, as1, ad1)
            return carry

        lax.fori_loop(0, T, pair, 0)
        scwait(as1)
        plsc.subcore_barrier()
        pltpu.sync_copy(acc.at[pl.ds(s * NT, NT)],
                        part.at[c, pl.ds(s * NT, NT)])

    return pl.kernel(
        body,
        out_type=[jax.ShapeDtypeStruct((E + NW * CG, D), F32),
                  jax.ShapeDtypeStruct((NC, NP, D), F32)],
        mesh=_sc_mesh(),
        scratch_types=[
            pltpu.VMEM((16, CG), jnp.int32),
            pltpu.VMEM((16, CG), jnp.int32),
            pltpu.VMEM((CG, D), F32),
            pltpu.VMEM((CG, D), F32),
            pltpu.VMEM((CG, D), F32),
            pltpu.VMEM((CG, D), F32),
            pltpu.VMEM_SHARED((NP, D), F32),
            pltpu.SemaphoreType.DMA,
            pltpu.SemaphoreType.DMA,
            pltpu.SemaphoreType.DMA,
        ],
    )


def _make_segsum(N, E, D):
    EW = E // NW
    QN = EW // CH
    NP, NT = _pad_rows(N)

    T = -(-QN // 2)

    def body(evals, dst3, part, dst_v, e0, e1, acc, sr0, sr1, sw0, sw1):
        c = lax.axis_index("c")
        s = lax.axis_index("s")
        wid = c * NS + s
        _fill_zero(e0, CH, D)
        _zero_acc_rows(e0, CH, acc, s * NT, NT)
        plsc.subcore_barrier()

        def rd(q, buf, sem):
            pltpu.async_copy(evals.at[pl.ds(wid * EW + q * CH, CH)], buf, sem)

        def dma_wait(buf, sem):
            pltpu.make_async_copy(evals.at[pl.ds(0, CH)], buf, sem).wait()

        def sc(k, buf, sem):
            pltpu.async_copy(buf, acc.at[dst_v.at[k]], sem, add=True)

        rd(0, e0, sr0)

        def pair(t, carry):
            q0 = 2 * t
            q1 = q0 + 1

            @pl.when(t > 0)
            def _():
                dma_wait(e1, sw1)

            @pl.when(lax.rem(t, 4) == 0)
            def _():
                pltpu.sync_copy(dst3.at[wid, pl.ds((t // 4) * 8, 8)], dst_v)

            @pl.when(q1 < QN)
            def _():
                rd(q1, e1, sr1)

            dma_wait(e0, sr0)
            sc(lax.rem(q0, 8), e0, sw0)

            @pl.when(q1 < QN)
            def _():
                dma_wait(e1, sr1)

            dma_wait(e0, sw0)

            @pl.when(q0 + 2 < QN)
            def _():
                rd(q0 + 2, e0, sr0)

            @pl.when(q1 < QN)
            def _():
                sc(lax.rem(q1, 8), e1, sw1)

            return carry

        lax.fori_loop(0, T, pair, 0)
        if QN % 2 == 0:
            dma_wait(e1, sw1)
        plsc.subcore_barrier()
        pltpu.sync_copy(acc.at[pl.ds(s * NT, NT)],
                        part.at[c, pl.ds(s * NT, NT)])

    return pl.kernel(
        body,
        out_type=jax.ShapeDtypeStruct((NC, NP, D), F32),
        mesh=_sc_mesh(),
        scratch_types=[
            pltpu.VMEM((8, CH), jnp.int32),
            pltpu.VMEM((CH, D), F32),
            pltpu.VMEM((CH, D), F32),
            pltpu.VMEM_SHARED((NP, D), F32),
            pltpu.SemaphoreType.DMA,
            pltpu.SemaphoreType.DMA,
            pltpu.SemaphoreType.DMA,
            pltpu.SemaphoreType.DMA,
        ],
    )


def _make_edge_mlp(E, D, H):
    NB = E // BE

    def body(sumh_ref, bond_ref, w1a, w1b, b1, g, bt, w2, b2, e_ref,
             s1, s2, ca, cb):
        p = pl.program_id(0)
        j = pl.program_id(1)
        bf = jnp.bfloat16
        y = (jnp.dot(sumh_ref[...].astype(bf), w1a[...].astype(bf),
                     preferred_element_type=F32)
             + jnp.dot(bond_ref[...].astype(bf), w1b[...].astype(bf),
                       preferred_element_type=F32)
             + b1[...])

        @pl.when(p == 0)
        def _():
            @pl.when(j == 0)
            def _():
                s1[...] = jnp.zeros_like(s1)
                s2[...] = jnp.zeros_like(s2)

            s1[...] += jnp.sum(y, axis=0, keepdims=True)
            s2[...] += jnp.sum(y * y, axis=0, keepdims=True)

            @pl.when(j == NB - 1)
            def _():
                mu = s1[...] / E
                var = s2[...] / E - mu * mu
                a = g[...] * lax.rsqrt(var + EPS)
                ca[...] = a
                cb[...] = bt[...] - mu * a

        @pl.when(p == 1)
        def _():
            yh = jnp.maximum(y * ca[...] + cb[...], 0.0)
            e_ref[...] = jnp.dot(yh.astype(bf), w2[...].astype(bf),
                                 preferred_element_type=F32) + b2[...]

    return pl.pallas_call(
        body,
        grid=(2, NB),
        in_specs=[
            pl.BlockSpec((BE, D), lambda p, j: (j, 0)),
            pl.BlockSpec((BE, D), lambda p, j: (j, 0)),
            pl.BlockSpec((D, H), lambda p, j: (0, 0)),
            pl.BlockSpec((D, H), lambda p, j: (0, 0)),
            pl.BlockSpec((1, H), lambda p, j: (0, 0)),
            pl.BlockSpec((1, H), lambda p, j: (0, 0)),
            pl.BlockSpec((1, H), lambda p, j: (0, 0)),
            pl.BlockSpec((H, D), lambda p, j: (0, 0)),
            pl.BlockSpec((1, D), lambda p, j: (0, 0)),
        ],
        out_specs=pl.BlockSpec((BE, D), lambda p, j: (jnp.where(p == 0, 0, j), 0)),
        out_shape=jax.ShapeDtypeStruct((E, D), F32),
        scratch_shapes=[pltpu.VMEM((1, H), F32)] * 4,
        compiler_params=pltpu.CompilerParams(
            dimension_semantics=("arbitrary", "arbitrary")),
    )


def _make_node_mlp(N, D, H):
    def body(ph, pe, w1a, w1b, b1, g, bt, w2, b2, h_ref):
        bf = jnp.bfloat16
        sh = ph[0, :N] + ph[1, :N]
        se = pe[0, :N] + pe[1, :N]
        y = (jnp.dot(sh.astype(bf), w1a[...].astype(bf),
                     preferred_element_type=F32)
             + jnp.dot(se.astype(bf), w1b[...].astype(bf),
                       preferred_element_type=F32)
             + b1[...])
        mu = jnp.mean(y, axis=0, keepdims=True)
        var = jnp.mean(y * y, axis=0, keepdims=True) - mu * mu
        yh = jnp.maximum((y - mu) * (g[...] * lax.rsqrt(var + EPS)) + bt[...], 0.0)
        h_ref[...] = jnp.dot(yh.astype(bf), w2[...].astype(bf),
                             preferred_element_type=F32) + b2[...]

    return pl.pallas_call(
        body,
        out_shape=jax.ShapeDtypeStruct((N, D), F32),
    )


def kernel(atom, bond, edge_index, Wa1, ba1, ga, bta, Wa2, ba2,
           Wb1, bb1, gb, btb, Wb2, bb2):
    N, D = atom.shape
    E = bond.shape[0]
    H = Wb1.shape[1]
    assert E % (NW * CH) == 0 and E % BE == 0 and E % CG == 0
    QN = E // (NW * CH)
    pad = -(-QN // 8) * 8 - QN

    dst3 = jnp.pad(edge_index[1].reshape(NW, QN, CH), ((0, 0), (0, pad), (0, 0)))

    _, QMAIN, XTRA, _, NQPAD = _gather_geometry(N, E)
    nmain = QMAIN * NW * CG
    src_m = edge_index[0, :nmain].reshape(NW, QMAIN, CG)
    dst_m = edge_index[1, :nmain].reshape(NW, QMAIN, CG)
    tail_src = jnp.zeros((NW, NQPAD - QMAIN, CG), jnp.int32)
    tail_dst = jnp.full((NW, NQPAD - QMAIN, CG), N, jnp.int32)
    if XTRA:
        tail_src = tail_src.at[:XTRA, 0].set(
            edge_index[0, nmain:].reshape(XTRA, CG))
        tail_dst = tail_dst.at[:XTRA, 0].set(
            edge_index[1, nmain:].reshape(XTRA, CG))
    src4 = jnp.concatenate([src_m, tail_src], axis=1)
    dst4 = jnp.concatenate([dst_m, tail_dst], axis=1)

    sumh, ph = _make_gather_segsum(N, E, D)(atom, src4, dst4)
    e = _make_edge_mlp(E, D, H)(
        sumh, bond, Wb1[:D], Wb1[D:], bb1.reshape(1, H), gb.reshape(1, H),
        btb.reshape(1, H), Wb2, bb2.reshape(1, D))
    pe = _make_segsum(N, E, D)(e, dst3)
    h = _make_node_mlp(N, D, H)(
        ph, pe, Wa1[:D], Wa1[D:], ba1.reshape(1, H), ga.reshape(1, H),
        bta.reshape(1, H), Wa2, ba2.reshape(1, D))
    return h, e


# speedup vs baseline: 1.1557x; 1.0082x over previous
import jax
import jax.numpy as jnp
from jax import lax
from jax.experimental import pallas as pl
from jax.experimental.pallas import tpu as pltpu
from jax.experimental.pallas import tpu_sc as plsc

F32 = jnp.float32
EPS = 1e-5
NC = 2
NS = 16
NW = NC * NS
LANES = 16
CH = 80
BE = 10000


def _sc_mesh():
    return plsc.VectorSubcoreMesh(
        core_axis_name="c", subcore_axis_name="s",
        num_cores=NC, num_subcores=NS)


def _fill_zero(buf, rows, cols):
    zero = jnp.zeros((LANES,), F32)

    def body(r, carry):
        for j in range(cols // LANES):
            buf[r, pl.ds(j * LANES, LANES)] = zero
        return carry

    lax.fori_loop(0, rows, body, 0)


def _zero_acc_rows(src_buf, rows, acc, base, nrows):
    nfull, rem = divmod(nrows, rows)
    for k in range(nfull):
        pltpu.sync_copy(src_buf, acc.at[pl.ds(base + k * rows, rows)])
    if rem:
        pltpu.sync_copy(src_buf.at[pl.ds(0, rem)],
                        acc.at[pl.ds(base + nfull * rows, rem)])


def _pad_rows(N):
    nt = -(-N // NS)
    nt = -(-nt // 8) * 8
    return nt * NS, nt


CG = 64


def _gather_geometry(N, E):
    QTOT = E // CG
    QMAIN = QTOT // NW
    XTRA = QTOT - QMAIN * NW
    NQ = QMAIN + 1
    NQ += NQ % 2
    NQPAD = -(-NQ // 8) * 8
    return QTOT, QMAIN, XTRA, NQ, NQPAD


def _make_gather_segsum(N, E, D):
    QTOT, QMAIN, XTRA, NQ, NQPAD = _gather_geometry(N, E)
    NP, NT = _pad_rows(N)
    T = NQ // 2
    MW = QMAIN * CG

    def body(atom, src3, dst3, sumh, part, src_v, dst_v,
             as0, ad0, as1, ad1, acc, sg0, sg1, swc):
        c = lax.axis_index("c")
        s = lax.axis_index("s")
        wid = c * NS + s
        _fill_zero(ad0, CG, D)
        _zero_acc_rows(ad0, CG, acc, s * NT, NT)
        plsc.subcore_barrier()

        def ebase(q):
            main = wid * MW + q * CG
            extra = QMAIN * NW * CG + wid * CG
            sink = E + wid * CG
            is_extra = jnp.logical_and(q == QMAIN, wid < XTRA)
            return jnp.where(q < QMAIN, main, jnp.where(is_extra, extra, sink))

        def stage(g):
            h = lax.rem(g, 2) * 8
            pltpu.sync_copy(src3.at[wid, pl.ds(g * 8, 8)], src_v.at[pl.ds(h, 8)])
            pltpu.sync_copy(dst3.at[wid, pl.ds(g * 8, 8)], dst_v.at[pl.ds(h, 8)])

        def gath(q, bs, bd, sem):
            k = lax.rem(q, 16)
            pltpu.async_copy(atom.at[src_v.at[k]], bs, sem)
            pltpu.async_copy(atom.at[dst_v.at[k]], bd, sem)

        def pair_wait(bs, bd, sem):
            pltpu.make_async_copy(atom.at[pl.ds(0, CG)], bs, sem).wait()
            pltpu.make_async_copy(atom.at[pl.ds(0, CG)], bd, sem).wait()

        def wr(q, bs, bd):
            pltpu.sync_copy(bd, sumh.at[pl.ds(ebase(q), CG)])
            pltpu.async_copy(bs, acc.at[dst_v.at[lax.rem(q, 16)]], swc, add=True)

        def scwait(buf):
            pltpu.make_async_copy(atom.at[pl.ds(0, CG)], buf, swc).wait()

        def comp(bs, bd):
            def row(r, carry):
                for j in range(D // LANES):
                    sl = pl.ds(j * LANES, LANES)
                    bd[r, sl] = bs[r, sl] + bd[r, sl]
                return carry

            lax.fori_loop(0, CG, row, 0)

        stage(0)
        gath(0, as0, ad0, sg0)

        def pair(t, carry):
            q0 = 2 * t
            q1 = q0 + 1

            @pl.when(t > 0)
            def _():
                scwait(as1)

            gath(q1, as1, ad1, sg1)
            pair_wait(as0, ad0, sg0)
            comp(as0, ad0)
            wr(q0, as0, ad0)
            pair_wait(as1, ad1, sg1)
            comp(as1, ad1)
            scwait(as0)

            @pl.when(jnp.logical_and(lax.rem(q0 + 2, 8) == 0, t + 1 < T))
            def _():
                stage((q0 + 2) // 8)

            @pl.when(t + 1 < T)
            def _():
                gath(q0 + 2, as0, ad0, sg0)

            wr(q1, as1, ad1)
            return carry

        lax.fori_loop(0, T, pair, 0)
        scwait(as1)
        plsc.subcore_barrier()
        pltpu.sync_copy(acc.at[pl.ds(s * NT, NT)],
                        part.at[c, pl.ds(s * NT, NT)])

    return pl.kernel(
        body,
        out_type=[jax.ShapeDtypeStruct((E + NW * CG, D), F32),
                  jax.ShapeDtypeStruct((NC, NP, D), F32)],
        mesh=_sc_mesh(),
        scratch_types=[
            pltpu.VMEM((16, CG), jnp.int32),
            pltpu.VMEM((16, CG), jnp.int32),
            pltpu.VMEM((CG, D), F32),
            pltpu.VMEM((CG, D), F32),
            pltpu.VMEM((CG, D), F32),
            pltpu.VMEM((CG, D), F32),
            pltpu.VMEM_SHARED((NP, D), F32),
            pltpu.SemaphoreType.DMA,
            pltpu.SemaphoreType.DMA,
            pltpu.SemaphoreType.DMA,
        ],
    )


def _make_segsum(N, E, D):
    EW = E // NW
    QN = EW // CH
    NP, NT = _pad_rows(N)

    T = -(-QN // 2)

    def body(evals, dst3, part, dst_v, e0, e1, acc, sr0, sr1, sw0, sw1):
        c = lax.axis_index("c")
        s = lax.axis_index("s")
        wid = c * NS + s
        _fill_zero(e0, CH, D)
        _zero_acc_rows(e0, CH, acc, s * NT, NT)
        plsc.subcore_barrier()

        def rd(q, buf, sem):
            pltpu.async_copy(evals.at[pl.ds(wid * EW + q * CH, CH)], buf, sem)

        def dma_wait(buf, sem):
            pltpu.make_async_copy(evals.at[pl.ds(0, CH)], buf, sem).wait()

        def sc(k, buf, sem):
            pltpu.async_copy(buf, acc.at[dst_v.at[k]], sem, add=True)

        rd(0, e0, sr0)

        def pair(t, carry):
            q0 = 2 * t
            q1 = q0 + 1

            @pl.when(t > 0)
            def _():
                dma_wait(e1, sw1)

            @pl.when(lax.rem(t, 4) == 0)
            def _():
                pltpu.sync_copy(dst3.at[wid, pl.ds((t // 4) * 8, 8)], dst_v)

            @pl.when(q1 < QN)
            def _():
                rd(q1, e1, sr1)

            dma_wait(e0, sr0)
            sc(lax.rem(q0, 8), e0, sw0)

            @pl.when(q1 < QN)
            def _():
                dma_wait(e1, sr1)

            dma_wait(e0, sw0)

            @pl.when(q0 + 2 < QN)
            def _():
                rd(q0 + 2, e0, sr0)

            @pl.when(q1 < QN)
            def _():
                sc(lax.rem(q1, 8), e1, sw1)

            return carry

        lax.fori_loop(0, T, pair, 0)
        if QN % 2 == 0:
            dma_wait(e1, sw1)
        plsc.subcore_barrier()
        pltpu.sync_copy(acc.at[pl.ds(s * NT, NT)],
                        part.at[c, pl.ds(s * NT, NT)])

    return pl.kernel(
        body,
        out_type=jax.ShapeDtypeStruct((NC, NP, D), F32),
        mesh=_sc_mesh(),
        scratch_types=[
            pltpu.VMEM((8, CH), jnp.int32),
            pltpu.VMEM((CH, D), F32),
            pltpu.VMEM((CH, D), F32),
            pltpu.VMEM_SHARED((NP, D), F32),
            pltpu.SemaphoreType.DMA,
            pltpu.SemaphoreType.DMA,
            pltpu.SemaphoreType.DMA,
            pltpu.SemaphoreType.DMA,
        ],
    )


def _make_edge_mlp(E, D, H):
    NB = E // BE

    def body(sumh_ref, bond_ref, w1a, w1b, b1, g, bt, w2, b2, e_ref,
             s1, s2, ca, cb):
        p = pl.program_id(0)
        j = pl.program_id(1)
        bf = jnp.bfloat16
        y = (jnp.dot(sumh_ref[...].astype(bf), w1a[...].astype(bf),
                     preferred_element_type=F32)
             + jnp.dot(bond_ref[...].astype(bf), w1b[...].astype(bf),
                       preferred_element_type=F32)
             + b1[...])

        @pl.when(p == 0)
        def _():
            @pl.when(j == 0)
            def _():
                s1[...] = jnp.zeros_like(s1)
                s2[...] = jnp.zeros_like(s2)

            s1[...] += jnp.sum(y, axis=0, keepdims=True)
            s2[...] += jnp.sum(y * y, axis=0, keepdims=True)

            @pl.when(j == NB - 1)
            def _():
                mu = s1[...] / E
                var = s2[...] / E - mu * mu
                a = g[...] * lax.rsqrt(var + EPS)
                ca[...] = a
                cb[...] = bt[...] - mu * a

        @pl.when(p == 1)
        def _():
            yh = jnp.maximum(y * ca[...] + cb[...], 0.0)
            e_ref[...] = jnp.dot(yh.astype(bf), w2[...].astype(bf),
                                 preferred_element_type=F32) + b2[...]

    return pl.pallas_call(
        body,
        grid=(2, NB),
        in_specs=[
            pl.BlockSpec((BE, D), lambda p, j: (j, 0)),
            pl.BlockSpec((BE, D), lambda p, j: (j, 0)),
            pl.BlockSpec((D, H), lambda p, j: (0, 0)),
            pl.BlockSpec((D, H), lambda p, j: (0, 0)),
            pl.BlockSpec((1, H), lambda p, j: (0, 0)),
            pl.BlockSpec((1, H), lambda p, j: (0, 0)),
            pl.BlockSpec((1, H), lambda p, j: (0, 0)),
            pl.BlockSpec((H, D), lambda p, j: (0, 0)),
            pl.BlockSpec((1, D), lambda p, j: (0, 0)),
        ],
        out_specs=pl.BlockSpec((BE, D), lambda p, j: (jnp.where(p == 0, 0, j), 0)),
        out_shape=jax.ShapeDtypeStruct((E, D), F32),
        scratch_shapes=[pltpu.VMEM((1, H), F32)] * 4,
        compiler_params=pltpu.CompilerParams(
            dimension_semantics=("arbitrary", "arbitrary")),
    )


def _make_node_mlp(N, D, H):
    def body(ph, pe, w1a, w1b, b1, g, bt, w2, b2, h_ref):
        bf = jnp.bfloat16
        sh = ph[0, :N] + ph[1, :N]
        se = pe[0, :N] + pe[1, :N]
        y = (jnp.dot(sh.astype(bf), w1a[...].astype(bf),
                     preferred_element_type=F32)
             + jnp.dot(se.astype(bf), w1b[...].astype(bf),
                       preferred_element_type=F32)
             + b1[...])
        mu = jnp.mean(y, axis=0, keepdims=True)
        var = jnp.mean(y * y, axis=0, keepdims=True) - mu * mu
        yh = jnp.maximum((y - mu) * (g[...] * lax.rsqrt(var + EPS)) + bt[...], 0.0)
        h_ref[...] = jnp.dot(yh.astype(bf), w2[...].astype(bf),
                             preferred_element_type=F32) + b2[...]

    return pl.pallas_call(
        body,
        out_shape=jax.ShapeDtypeStruct((N, D), F32),
    )


def kernel(atom, bond, edge_index, Wa1, ba1, ga, bta, Wa2, ba2,
           Wb1, bb1, gb, btb, Wb2, bb2):
    N, D = atom.shape
    E = bond.shape[0]
    H = Wb1.shape[1]
    assert E % (NW * CH) == 0 and E % BE == 0 and E % CG == 0
    QN = E // (NW * CH)
    pad = -(-QN // 8) * 8 - QN

    dst3 = jnp.pad(edge_index[1].reshape(NW, QN, CH), ((0, 0), (0, pad), (0, 0)))

    _, QMAIN, XTRA, _, NQPAD = _gather_geometry(N, E)
    nmain = QMAIN * NW * CG
    src_m = edge_index[0, :nmain].reshape(NW, QMAIN, CG)
    dst_m = edge_index[1, :nmain].reshape(NW, QMAIN, CG)
    tail_src = jnp.zeros((NW, NQPAD - QMAIN, CG), jnp.int32)
    tail_dst = jnp.full((NW, NQPAD - QMAIN, CG), N, jnp.int32)
    if XTRA:
        tail_src = tail_src.at[:XTRA, 0].set(
            edge_index[0, nmain:].reshape(XTRA, CG))
        tail_dst = tail_dst.at[:XTRA, 0].set(
            edge_index[1, nmain:].reshape(XTRA, CG))
    src4 = jnp.concatenate([src_m, tail_src], axis=1)
    dst4 = jnp.concatenate([dst_m, tail_dst], axis=1)

    sumh, ph = _make_gather_segsum(N, E, D)(atom, src4, dst4)
    e = _make_edge_mlp(E, D, H)(
        sumh, bond, Wb1[:D], Wb1[D:], bb1.reshape(1, H), gb.reshape(1, H),
        btb.reshape(1, H), Wb2, bb2.reshape(1, D))
    pe = _make_segsum(N, E, D)(e, dst3)
    h = _make_node_mlp(N, D, H)(
        ph, pe, Wa1[:D], Wa1[D:], ba1.reshape(1, H), ga.reshape(1, H),
        bta.reshape(1, H), Wa2, ba2.reshape(1, D))
    return h, e
